# baseline (speedup 1.0000x reference)
import functools

import jax
import jax.numpy as jnp
from jax import lax
from jax.experimental import pallas as pl
from jax.experimental.pallas import tpu as pltpu
from jax.experimental.pallas import tpu_sc as plsc

_N = 10000
_E = 320000
_HID = 16
_NC = 2
_NS = 16
_NW = _NC * _NS
_CHUNK = 128
_CPW = 80
_EPW = _CHUNK * _CPW
_EPAD = _NW * _EPW
_NBUF = 8
_RPT = 640
_ACC_ROWS = _NS * _RPT
_PK = 8
_NP = _N // _PK
_ACCP = _ACC_ROWS // _PK


def _sc_segsum_body(p_hbm, src_hbm, dst_hbm, out_hbm, sidx, didx, rows, zrows,
                    acc, gsem, ssem):
    cid = lax.axis_index("c")
    sid = lax.axis_index("s")
    wid = cid * _NS + sid
    pltpu.sync_copy(src_hbm.at[pl.ds(wid * _CPW, _CPW)], sidx)
    pltpu.sync_copy(dst_hbm.at[pl.ds(wid * _CPW, _CPW)], didx)
    for r in range(_CHUNK):
        zrows[r, :] = jnp.zeros((_HID,), jnp.float32)
    for k in range(_RPT // _CHUNK):
        pltpu.sync_copy(zrows, acc.at[pl.ds(sid * _RPT + k * _CHUNK, _CHUNK)])
    plsc.subcore_barrier()

    for b in range(_NBUF):
        pltpu.async_copy(p_hbm.at[sidx.at[b]], rows[b], gsem[b])

    @pl.loop(0, _CPW // _NBUF)
    def _edges(g):
        for b in range(_NBUF):
            j = g * _NBUF + b
            pltpu.make_async_copy(p_hbm.at[sidx.at[j]], rows[b], gsem[b]).wait()
            pltpu.async_copy(rows[b], acc.at[didx.at[j]], ssem[b], add=True)
        for b in range(_NBUF):
            j = g * _NBUF + b
            pltpu.make_async_copy(rows[b], acc.at[didx.at[j]], ssem[b]).wait()
            jn = j + _NBUF

            @pl.when(jn < _CPW)
            def _prefetch():
                pltpu.async_copy(p_hbm.at[sidx.at[jn]], rows[b], gsem[b])

    plsc.subcore_barrier()
    pltpu.sync_copy(
        acc.at[pl.ds(sid * _RPT, _RPT)],
        out_hbm.at[pl.ds(cid * _ACC_ROWS + sid * _RPT, _RPT)])


_sc_segsum = functools.partial(
    pl.kernel,
    out_type=jax.ShapeDtypeStruct((_NC * _ACC_ROWS, _HID), jnp.float32),
    mesh=plsc.VectorSubcoreMesh(core_axis_name="c", subcore_axis_name="s",
                                num_cores=_NC, num_subcores=_NS),
    scratch_types=[
        pltpu.VMEM((_CPW, _CHUNK), jnp.int32),
        pltpu.VMEM((_CPW, _CHUNK), jnp.int32),
        [pltpu.VMEM((_CHUNK, _HID), jnp.float32) for _ in range(_NBUF)],
        pltpu.VMEM((_CHUNK, _HID), jnp.float32),
        pltpu.VMEM_SHARED((_ACC_ROWS, _HID), jnp.float32),
        [pltpu.SemaphoreType.DMA for _ in range(_NBUF)],
        [pltpu.SemaphoreType.DMA for _ in range(_NBUF)],
    ],
    compiler_params=pltpu.CompilerParams(use_tc_tiling_on_sc=False),
)(_sc_segsum_body)


def _tc_proj_body(x_ref, w_ref, o_ref):
    o_ref[...] = jnp.concatenate(
        [jnp.dot(x_ref[:, b, :], w_ref[...],
                 preferred_element_type=jnp.float32) for b in range(_PK)],
        axis=1)


def _tc_layer_body(p_ref, a_ref, b1_ref, w2_ref, b2_ref, w1n_ref, o_ref):
    agg = a_ref[0:_NP, :] + a_ref[_ACCP:_ACCP + _NP, :]
    z = jnp.maximum(p_ref[...] + agg + b1_ref[...], 0.0)
    t = jnp.dot(z, w2_ref[...], preferred_element_type=jnp.float32) + b2_ref[...]
    o_ref[...] = jnp.dot(t, w1n_ref[...], preferred_element_type=jnp.float32)


def _tc_head_body(p_ref, a_ref, b1_ref, w2_ref, b2_ref, wo1_ref, bo1_ref,
                  wo2_ref, bo2_ref, o_ref):
    agg = a_ref[0:_NP, :] + a_ref[_ACCP:_ACCP + _NP, :]
    z = jnp.maximum(p_ref[...] + agg + b1_ref[...], 0.0)
    h = jnp.dot(z, w2_ref[...], preferred_element_type=jnp.float32) + b2_ref[...]
    o = jnp.maximum(jnp.dot(h, wo1_ref[...],
                            preferred_element_type=jnp.float32) + bo1_ref[...], 0.0)
    o_ref[...] = jnp.dot(o, wo2_ref[...],
                         preferred_element_type=jnp.float32) + bo2_ref[...]


def kernel(X, edge_index, edge_weights, params):
    del edge_weights

    pad = _EPAD - _E
    ar = lax.iota(jnp.int32, pad)
    src_pad = ar % _N
    dst_pad = _N + ar % (_ACC_ROWS - _N)
    src2d = jnp.concatenate([edge_index[0], src_pad]).reshape(_NW * _CPW, _CHUNK)
    dst2d = jnp.concatenate([edge_index[1], dst_pad]).reshape(_NW * _CPW, _CHUNK)

    L = 5
    eye = jnp.eye(_PK, dtype=jnp.float32)
    kr = lambda w: jnp.kron(eye, w)
    tile = lambda b: jnp.tile(b, _PK).reshape(1, -1)
    b1 = [tile(params[f"b1_{i}"]) for i in range(L)]
    b2 = [tile(params[f"b2_{i}"]) for i in range(L)]
    w2 = [kr(params[f"W2_{i}"]) for i in range(L)]
    w1n = [kr(params[f"W1_{i}"]) for i in range(1, L)]
    wo1 = kr(params["Wo1"])
    bo1 = tile(params["bo1"])
    wo2 = kr(params["Wo2"])
    bo2 = jnp.tile(params["bo2"], _PK).reshape(1, _PK)

    f32 = jnp.float32
    pkd = jax.ShapeDtypeStruct((_NP, _PK * _HID), f32)
    p = pl.pallas_call(_tc_proj_body, out_shape=pkd)(
        X.reshape(_NP, _PK, 128), params["W1_0"])

    for i in range(L - 1):
        apart = _sc_segsum(p.reshape(_N, _HID), src2d, dst2d)
        p = pl.pallas_call(_tc_layer_body, out_shape=pkd)(
            p, apart.reshape(_NC * _ACCP, _PK * _HID), b1[i], w2[i], b2[i],
            w1n[i])

    apart = _sc_segsum(p.reshape(_N, _HID), src2d, dst2d)
    outp = pl.pallas_call(
        _tc_head_body,
        out_shape=jax.ShapeDtypeStruct((_NP, _PK), f32),
    )(p, apart.reshape(_NC * _ACCP, _PK * _HID), b1[L - 1], w2[L - 1],
      b2[L - 1], wo1, bo1, wo2, bo2)
    return outp.reshape(_N, 1)


# speedup vs baseline: 1.0038x; 1.0038x over previous
import functools

import jax
import jax.numpy as jnp
from jax import lax
from jax.experimental import pallas as pl
from jax.experimental.pallas import tpu as pltpu
from jax.experimental.pallas import tpu_sc as plsc

_N = 10000
_E = 320000
_HID = 16
_NC = 2
_NS = 16
_NW = _NC * _NS
_CHUNK = 128
_CPW = 80
_EPW = _CHUNK * _CPW
_EPAD = _NW * _EPW
_NBUF = 8
_RPT = 640
_ACC_ROWS = _NS * _RPT
_PK = 8
_NP = _N // _PK
_ACCP = _ACC_ROWS // _PK


def _sc_segsum_body(p_hbm, src_hbm, dst_hbm, out_hbm, sidx, didx, rows, zrows,
                    acc, gsem, ssem):
    cid = lax.axis_index("c")
    sid = lax.axis_index("s")
    wid = cid * _NS + sid
    pltpu.sync_copy(src_hbm.at[pl.ds(wid * _CPW, _CPW)], sidx)
    pltpu.sync_copy(dst_hbm.at[pl.ds(wid * _CPW, _CPW)], didx)
    for r in range(_CHUNK):
        zrows[r, :] = jnp.zeros((_HID,), jnp.float32)
    for k in range(_RPT // _CHUNK):
        pltpu.sync_copy(zrows, acc.at[pl.ds(sid * _RPT + k * _CHUNK, _CHUNK)])
    plsc.subcore_barrier()

    for b in range(_NBUF):
        pltpu.async_copy(p_hbm.at[sidx.at[b]], rows[b], gsem[b])

    @pl.loop(0, _CPW // _NBUF)
    def _edges(g):
        for b in range(_NBUF):
            j = g * _NBUF + b
            pltpu.make_async_copy(p_hbm.at[sidx.at[j]], rows[b], gsem[b]).wait()
            pltpu.async_copy(rows[b], acc.at[didx.at[j]], ssem[b], add=True)
        for b in range(_NBUF):
            j = g * _NBUF + b
            pltpu.make_async_copy(rows[b], acc.at[didx.at[j]], ssem[b]).wait()
            jn = j + _NBUF

            @pl.when(jn < _CPW)
            def _prefetch():
                pltpu.async_copy(p_hbm.at[sidx.at[jn]], rows[b], gsem[b])

    plsc.subcore_barrier()
    pltpu.sync_copy(
        acc.at[pl.ds(sid * _RPT, _RPT)],
        out_hbm.at[pl.ds(cid * _ACC_ROWS + sid * _RPT, _RPT)])


_sc_segsum = functools.partial(
    pl.kernel,
    out_type=jax.ShapeDtypeStruct((_NC * _ACC_ROWS, _HID), jnp.float32),
    mesh=plsc.VectorSubcoreMesh(core_axis_name="c", subcore_axis_name="s",
                                num_cores=_NC, num_subcores=_NS),
    scratch_types=[
        pltpu.VMEM((_CPW, _CHUNK), jnp.int32),
        pltpu.VMEM((_CPW, _CHUNK), jnp.int32),
        [pltpu.VMEM((_CHUNK, _HID), jnp.float32) for _ in range(_NBUF)],
        pltpu.VMEM((_CHUNK, _HID), jnp.float32),
        pltpu.VMEM_SHARED((_ACC_ROWS, _HID), jnp.float32),
        [pltpu.SemaphoreType.DMA for _ in range(_NBUF)],
        [pltpu.SemaphoreType.DMA for _ in range(_NBUF)],
    ],
    compiler_params=pltpu.CompilerParams(use_tc_tiling_on_sc=False),
)(_sc_segsum_body)


def _tc_proj_body(x_ref, w_ref, o_ref):
    o_ref[...] = jnp.concatenate(
        [jnp.dot(x_ref[:, b, :], w_ref[...],
                 preferred_element_type=jnp.float32) for b in range(_PK)],
        axis=1)


def _tc_layer0_body(p_ref, a_ref, b1_ref, w2_ref, b2_ref, o_ref):
    agg = a_ref[0:_NP, :] + a_ref[_ACCP:_ACCP + _NP, :]
    z = jnp.maximum(p_ref[...] + agg + b1_ref[...], 0.0)
    o_ref[...] = jnp.dot(z, w2_ref[...],
                         preferred_element_type=jnp.float32) + b2_ref[...]


def _tc_layer_body(h_ref, a_ref, w1_ref, b1_ref, w2_ref, b2_ref, o_ref):
    agg = a_ref[0:_NP, :] + a_ref[_ACCP:_ACCP + _NP, :]
    s = h_ref[...] + agg
    z = jnp.maximum(jnp.dot(s, w1_ref[...],
                            preferred_element_type=jnp.float32) + b1_ref[...], 0.0)
    o_ref[...] = jnp.dot(z, w2_ref[...],
                         preferred_element_type=jnp.float32) + b2_ref[...]


def _tc_head_body(h_ref, a_ref, w1_ref, b1_ref, w2_ref, b2_ref, wo1_ref,
                  bo1_ref, wo2_ref, bo2_ref, o_ref):
    agg = a_ref[0:_NP, :] + a_ref[_ACCP:_ACCP + _NP, :]
    s = h_ref[...] + agg
    z = jnp.maximum(jnp.dot(s, w1_ref[...],
                            preferred_element_type=jnp.float32) + b1_ref[...], 0.0)
    h = jnp.dot(z, w2_ref[...], preferred_element_type=jnp.float32) + b2_ref[...]
    o = jnp.maximum(jnp.dot(h, wo1_ref[...],
                            preferred_element_type=jnp.float32) + bo1_ref[...], 0.0)
    o_ref[...] = jnp.dot(o, wo2_ref[...],
                         preferred_element_type=jnp.float32) + bo2_ref[...]


def kernel(X, edge_index, edge_weights, params):
    del edge_weights

    pad = _EPAD - _E
    ar = lax.iota(jnp.int32, pad)
    src_pad = ar % _N
    dst_pad = _N + ar % (_ACC_ROWS - _N)
    src2d = jnp.concatenate([edge_index[0], src_pad]).reshape(_NW * _CPW, _CHUNK)
    dst2d = jnp.concatenate([edge_index[1], dst_pad]).reshape(_NW * _CPW, _CHUNK)

    L = 5
    eye = jnp.eye(_PK, dtype=jnp.float32)
    kr = lambda w: jnp.kron(eye, w)
    tile = lambda b: jnp.tile(b, _PK).reshape(1, -1)
    b1 = [tile(params[f"b1_{i}"]) for i in range(L)]
    b2 = [tile(params[f"b2_{i}"]) for i in range(L)]
    w1 = [kr(params[f"W1_{i}"]) for i in range(1, L)]
    w2 = [kr(params[f"W2_{i}"]) for i in range(L)]
    wo1 = kr(params["Wo1"])
    bo1 = tile(params["bo1"])
    wo2 = kr(params["Wo2"])
    bo2 = jnp.tile(params["bo2"], _PK).reshape(1, _PK)

    f32 = jnp.float32
    pkd = jax.ShapeDtypeStruct((_NP, _PK * _HID), f32)
    p = pl.pallas_call(_tc_proj_body, out_shape=pkd)(
        X.reshape(_NP, _PK, 128), params["W1_0"])

    apart = _sc_segsum(p.reshape(_N, _HID), src2d, dst2d)
    h = pl.pallas_call(_tc_layer0_body, out_shape=pkd)(
        p, apart.reshape(_NC * _ACCP, _PK * _HID), b1[0], w2[0], b2[0])

    for i in range(1, L - 1):
        apart = _sc_segsum(h.reshape(_N, _HID), src2d, dst2d)
        h = pl.pallas_call(_tc_layer_body, out_shape=pkd)(
            h, apart.reshape(_NC * _ACCP, _PK * _HID), w1[i - 1], b1[i],
            w2[i], b2[i])

    apart = _sc_segsum(h.reshape(_N, _HID), src2d, dst2d)
    outp = pl.pallas_call(
        _tc_head_body,
        out_shape=jax.ShapeDtypeStruct((_NP, _PK), f32),
    )(h, apart.reshape(_NC * _ACCP, _PK * _HID), w1[L - 2], b1[L - 1],
      w2[L - 1], b2[L - 1], wo1, bo1, wo2, bo2)
    return outp.reshape(_N, 1)
